# Initial kernel scaffold; baseline (speedup 1.0000x reference)
import jax
import jax.numpy as jnp
from jax.experimental import pallas as pl


def kernel(x, edge_index, batch, W1a, b1a, W1b, b1b, Wsc1, bsc1, W2a, b2a, W2b, b2b, Wsc2, bsc2, Wm, bm, Wo, bo):
    raise NotImplementedError("write your pallas kernel here")


# speedup vs baseline: 2.8072x; 2.8072x over previous
import functools

import jax
import jax.numpy as jnp
from jax import lax
from jax.experimental import pallas as pl
from jax.experimental.pallas import tpu as pltpu
from jax.experimental.pallas import tpu_sc as plsc

N = 10000
E = 320000
D = 128
G = 64

NB_N = 10
BN = N // NB_N
BE = 2000
NB_E = E // BE

NC = 2
NS = 16
NW = NC * NS
EPW = E // NW
CHUNK = 80
CPW = EPW // CHUNK
ROWS_MAIN = 624
CW = 32
TAIL0 = NS * ROWS_MAIN
TAIL = N - TAIL0


def _prep_body(x_ref, wu_ref, wv_ref, wsc_ref, bsc_ref, u_ref, v_ref, sc_ref):
    xb = x_ref[...]
    u_ref[...] = jnp.dot(xb, wu_ref[...], preferred_element_type=jnp.float32)
    v_ref[...] = jnp.dot(xb, wv_ref[...], preferred_element_type=jnp.float32)
    sc_ref[...] = (
        jnp.dot(xb, wsc_ref[...], preferred_element_type=jnp.float32)
        + bsc_ref[...]
    )


def _edge_body(pu_ref, pv_ref, ba_ref, wb_ref, bb_ref, t_ref):
    p = jnp.maximum(pu_ref[...] + pv_ref[...] + ba_ref[...], 0.0)
    t = jnp.dot(p, wb_ref[...], preferred_element_type=jnp.float32) + bb_ref[...]
    t_ref[...] = jnp.maximum(t, 0.0)


def _combine_body(sp_ref, cp_ref, scin_ref, batch_ref,
                  o_ref, s_ref, q_ref, c_ref):
    i = pl.program_id(0)

    @pl.when(i == 0)
    def _():
        s_ref[...] = jnp.zeros_like(s_ref)
        q_ref[...] = jnp.zeros_like(q_ref)
        c_ref[...] = jnp.zeros_like(c_ref)

    sp = sp_ref[...]
    cp = cp_ref[...]
    cnt = cp[0, :, 0:1] + cp[1, :, 0:1]
    agg = (sp[0] + sp[1]) / jnp.maximum(cnt, 1.0)
    o = agg + scin_ref[...]
    o_ref[...] = o

    b = batch_ref[0, 0, :]
    gid = lax.broadcasted_iota(jnp.int32, (BN, G), 1)
    oh = (b[:, None] == gid).astype(jnp.float32)
    dn = (((0,), (0,)), ((), ()))
    s_ref[...] += lax.dot_general(oh, o, dn, preferred_element_type=jnp.float32)
    q_ref[...] += lax.dot_general(oh, o * o, dn,
                                  preferred_element_type=jnp.float32)
    colsum = jnp.sum(oh, axis=0)
    c_ref[...] += jnp.broadcast_to(colsum[None, :], (8, G))


def _norm_stats(s_ref, q_ref, c_ref):
    cg = jnp.maximum(c_ref[0, :], 1.0)[:, None]
    mean = s_ref[...] / cg
    var = jnp.maximum(q_ref[...] / cg - mean * mean, 0.0)
    return mean, var


def _norm_rows(o, batch_row, mean, var):
    gid = lax.broadcasted_iota(jnp.int32, (o.shape[0], G), 1)
    oh = (batch_row[:, None] == gid).astype(jnp.float32)
    mean_r = jnp.dot(oh, mean, preferred_element_type=jnp.float32)
    var_r = jnp.dot(oh, var, preferred_element_type=jnp.float32)
    h = (o - mean_r) * lax.rsqrt(var_r + 1e-5)
    return jnp.maximum(h, 0.0), oh


def _norm_prep_body(o_ref, batch_ref, s_ref, q_ref, c_ref,
                    wu_ref, wv_ref, wsc_ref, bsc_ref,
                    u_ref, v_ref, sc_ref):
    mean, var = _norm_stats(s_ref, q_ref, c_ref)
    h, _ = _norm_rows(o_ref[...], batch_ref[0, 0, :], mean, var)
    u_ref[...] = jnp.dot(h, wu_ref[...], preferred_element_type=jnp.float32)
    v_ref[...] = jnp.dot(h, wv_ref[...], preferred_element_type=jnp.float32)
    sc_ref[...] = (
        jnp.dot(h, wsc_ref[...], preferred_element_type=jnp.float32)
        + bsc_ref[...]
    )


def _norm_pool_body(o_ref, batch_ref, s_ref, q_ref, c_ref, pool_ref):
    i = pl.program_id(0)

    @pl.when(i == 0)
    def _():
        pool_ref[...] = jnp.zeros_like(pool_ref)

    mean, var = _norm_stats(s_ref, q_ref, c_ref)
    h, oh = _norm_rows(o_ref[...], batch_ref[0, 0, :], mean, var)
    dn = (((0,), (0,)), ((), ()))
    pool_ref[...] += lax.dot_general(oh, h, dn,
                                     preferred_element_type=jnp.float32)


def _final_body(pool_ref, c_ref, wm_ref, bm_ref, wo_ref, bo_ref, out_ref):
    cg = jnp.maximum(c_ref[0, :], 1.0)[:, None]
    pooled = pool_ref[...] / cg
    h = jnp.maximum(
        jnp.dot(pooled, wm_ref[...], preferred_element_type=jnp.float32)
        + bm_ref[...], 0.0)
    out_ref[...] = (
        jnp.dot(h, wo_ref[...], preferred_element_type=jnp.float32)
        + bo_ref[...]
    )


_full = lambda *shape: pl.BlockSpec(shape, lambda i: tuple(0 for _ in shape))


def _prep(x, wu, wv, wsc, bsc):
    return pl.pallas_call(
        _prep_body,
        grid=(NB_N,),
        in_specs=[
            pl.BlockSpec((BN, D), lambda i: (i, 0)),
            _full(D, D), _full(D, D), _full(D, D), _full(1, D),
        ],
        out_specs=[pl.BlockSpec((BN, D), lambda i: (i, 0))] * 3,
        out_shape=[jax.ShapeDtypeStruct((N, D), jnp.float32)] * 3,
    )(x, wu, wv, wsc, bsc)


def _edge_mlp(pu, pv, ba, wb, bb):
    return pl.pallas_call(
        _edge_body,
        grid=(NB_E,),
        in_specs=[
            pl.BlockSpec((BE, D), lambda i: (i, 0)),
            pl.BlockSpec((BE, D), lambda i: (i, 0)),
            _full(1, D), _full(D, D), _full(1, D),
        ],
        out_specs=pl.BlockSpec((BE, D), lambda i: (i, 0)),
        out_shape=jax.ShapeDtypeStruct((E, D), jnp.float32),
    )(pu, pv, ba, wb, bb)


def _combine_stats(sparts, cparts, scin, batch3):
    return pl.pallas_call(
        _combine_body,
        grid=(NB_N,),
        in_specs=[
            pl.BlockSpec((2, BN, D), lambda i: (0, i, 0)),
            pl.BlockSpec((2, BN, CW), lambda i: (0, i, 0)),
            pl.BlockSpec((BN, D), lambda i: (i, 0)),
            pl.BlockSpec((1, 1, BN), lambda i: (i, 0, 0)),
        ],
        out_specs=[
            pl.BlockSpec((BN, D), lambda i: (i, 0)),
            _full(G, D), _full(G, D), _full(8, G),
        ],
        out_shape=[
            jax.ShapeDtypeStruct((N, D), jnp.float32),
            jax.ShapeDtypeStruct((G, D), jnp.float32),
            jax.ShapeDtypeStruct((G, D), jnp.float32),
            jax.ShapeDtypeStruct((8, G), jnp.float32),
        ],
    )(sparts, cparts, scin, batch3)


def _norm_prep(o, batch3, s, q, c, wu, wv, wsc, bsc):
    return pl.pallas_call(
        _norm_prep_body,
        grid=(NB_N,),
        in_specs=[
            pl.BlockSpec((BN, D), lambda i: (i, 0)),
            pl.BlockSpec((1, 1, BN), lambda i: (i, 0, 0)),
            _full(G, D), _full(G, D), _full(8, G),
            _full(D, D), _full(D, D), _full(D, D), _full(1, D),
        ],
        out_specs=[pl.BlockSpec((BN, D), lambda i: (i, 0))] * 3,
        out_shape=[jax.ShapeDtypeStruct((N, D), jnp.float32)] * 3,
    )(o, batch3, s, q, c, wu, wv, wsc, bsc)


def _norm_pool(o, batch3, s, q, c):
    return pl.pallas_call(
        _norm_pool_body,
        grid=(NB_N,),
        in_specs=[
            pl.BlockSpec((BN, D), lambda i: (i, 0)),
            pl.BlockSpec((1, 1, BN), lambda i: (i, 0, 0)),
            _full(G, D), _full(G, D), _full(8, G),
        ],
        out_specs=_full(G, D),
        out_shape=jax.ShapeDtypeStruct((G, D), jnp.float32),
    )(o, batch3, s, q, c)


def _final_mlp(pool, c, wm, bm, wo, bo):
    return pl.pallas_call(
        _final_body,
        in_specs=[
            pl.BlockSpec((G, D), lambda: (0, 0)),
            pl.BlockSpec((8, G), lambda: (0, 0)),
            pl.BlockSpec((D, 256), lambda: (0, 0)),
            pl.BlockSpec((1, 256), lambda: (0, 0)),
            pl.BlockSpec((256, 2), lambda: (0, 0)),
            pl.BlockSpec((1, 2), lambda: (0, 0)),
        ],
        out_specs=pl.BlockSpec((G, 2), lambda: (0, 0)),
        out_shape=jax.ShapeDtypeStruct((G, 2), jnp.float32),
    )(pool, c, wm, bm, wo, bo)


@functools.cache
def _sc_mesh():
    return plsc.VectorSubcoreMesh(core_axis_name="c", subcore_axis_name="s")


@functools.cache
def _make_sc_gather():
    def body(u_hbm, v_hbm, dst_hbm, src_hbm, pu_hbm, pv_hbm,
             ibuf_d, ibuf_s, rows_u, rows_v, sem1, sem2):
        wid = lax.axis_index("s") * NC + lax.axis_index("c")

        def loop(k, _):
            base = wid * EPW + k * CHUNK
            pltpu.sync_copy(dst_hbm.at[pl.ds(base, CHUNK)], ibuf_d)
            pltpu.sync_copy(src_hbm.at[pl.ds(base, CHUNK)], ibuf_s)
            pltpu.async_copy(u_hbm.at[ibuf_d], rows_u, sem1).wait()
            pltpu.async_copy(v_hbm.at[ibuf_s], rows_v, sem2).wait()
            pltpu.sync_copy(rows_u, pu_hbm.at[pl.ds(base, CHUNK)])
            pltpu.sync_copy(rows_v, pv_hbm.at[pl.ds(base, CHUNK)])
            return 0

        lax.fori_loop(0, CPW, loop, 0)

    return pl.kernel(
        body,
        out_type=[
            jax.ShapeDtypeStruct((E, D), jnp.float32),
            jax.ShapeDtypeStruct((E, D), jnp.float32),
        ],
        mesh=_sc_mesh(),
        scratch_types=[
            pltpu.VMEM((CHUNK,), jnp.int32),
            pltpu.VMEM((CHUNK,), jnp.int32),
            pltpu.VMEM((CHUNK, D), jnp.float32),
            pltpu.VMEM((CHUNK, D), jnp.float32),
            pltpu.SemaphoreType.DMA,
            pltpu.SemaphoreType.DMA,
        ],
    )


def _sc_gather(u, v, dst, src):
    return _make_sc_gather()(u, v, dst, src)


@functools.cache
def _make_sc_scatter():
    def body(t_hbm, dst_hbm, z128, sparts, acc, ibuf, tbuf):
        cid = lax.axis_index("c")
        sid = lax.axis_index("s")
        wid = sid * NC + cid
        r0 = pl.multiple_of(sid * ROWS_MAIN, 8)

        pltpu.sync_copy(z128.at[pl.ds(r0, ROWS_MAIN)],
                        acc.at[pl.ds(r0, ROWS_MAIN)])

        @pl.when(sid == NS - 1)
        def _():
            pltpu.sync_copy(z128.at[pl.ds(TAIL0, TAIL)],
                            acc.at[pl.ds(TAIL0, TAIL)])

        plsc.subcore_barrier()

        def loop(k, _):
            base = wid * EPW + k * CHUNK
            pltpu.sync_copy(dst_hbm.at[pl.ds(base, CHUNK)], ibuf)
            pltpu.sync_copy(t_hbm.at[pl.ds(base, CHUNK)], tbuf)
            pltpu.sync_copy(tbuf, acc.at[ibuf], add=True)
            return 0

        lax.fori_loop(0, CPW, loop, 0)
        plsc.subcore_barrier()

        o0 = pl.multiple_of(cid * N + r0, 8)
        pltpu.sync_copy(acc.at[pl.ds(r0, ROWS_MAIN)],
                        sparts.at[pl.ds(o0, ROWS_MAIN)])

        @pl.when(sid == NS - 1)
        def _():
            ot = pl.multiple_of(cid * N + TAIL0, 8)
            pltpu.sync_copy(acc.at[pl.ds(TAIL0, TAIL)],
                            sparts.at[pl.ds(ot, TAIL)])

    return pl.kernel(
        body,
        out_type=[jax.ShapeDtypeStruct((NC * N, D), jnp.float32)],
        mesh=_sc_mesh(),
        scratch_types=[
            pltpu.VMEM_SHARED((N, D), jnp.float32),
            pltpu.VMEM((CHUNK,), jnp.int32),
            pltpu.VMEM((CHUNK, D), jnp.float32),
        ],
    )


@functools.cache
def _make_sc_count():
    def body(dst_hbm, z128, ones_hbm, cparts, acc, ibuf, ones_v):
        cid = lax.axis_index("c")
        sid = lax.axis_index("s")
        wid = sid * NC + cid
        r0 = pl.multiple_of(sid * ROWS_MAIN, 8)

        pltpu.sync_copy(z128.at[pl.ds(r0, ROWS_MAIN)],
                        acc.at[pl.ds(r0, ROWS_MAIN)])
        pltpu.sync_copy(ones_hbm, ones_v)

        @pl.when(sid == NS - 1)
        def _():
            pltpu.sync_copy(z128.at[pl.ds(TAIL0, TAIL)],
                            acc.at[pl.ds(TAIL0, TAIL)])

        plsc.subcore_barrier()

        def loop(k, _):
            base = wid * EPW + k * CHUNK
            pltpu.sync_copy(dst_hbm.at[pl.ds(base, CHUNK)], ibuf)
            pltpu.sync_copy(ones_v, acc.at[ibuf], add=True)
            return 0

        lax.fori_loop(0, CPW, loop, 0)
        plsc.subcore_barrier()

        o0 = pl.multiple_of(cid * N + r0, 8)
        pltpu.sync_copy(acc.at[pl.ds(r0, ROWS_MAIN)],
                        cparts.at[pl.ds(o0, ROWS_MAIN)])

        @pl.when(sid == NS - 1)
        def _():
            ot = pl.multiple_of(cid * N + TAIL0, 8)
            pltpu.sync_copy(acc.at[pl.ds(TAIL0, TAIL)],
                            cparts.at[pl.ds(ot, TAIL)])

    return pl.kernel(
        body,
        out_type=[jax.ShapeDtypeStruct((NC * N, D), jnp.float32)],
        mesh=_sc_mesh(),
        scratch_types=[
            pltpu.VMEM_SHARED((N, D), jnp.float32),
            pltpu.VMEM((CHUNK,), jnp.int32),
            pltpu.VMEM((CHUNK, D), jnp.float32),
        ],
    )


def _sc_count(dst, z128, ones128):
    (cparts,) = _make_sc_count()(dst, z128, ones128)
    return cparts.reshape(NC, N, D)[:, :, :CW]


def _sc_scatter(t, dst, z128):
    (sparts,) = _make_sc_scatter()(t, dst, z128)
    return sparts.reshape(NC, N, D)


def kernel(x, edge_index, batch, W1a, b1a, W1b, b1b, Wsc1, bsc1,
           W2a, b2a, W2b, b2b, Wsc2, bsc2, Wm, bm, Wo, bo):
    src = edge_index[0]
    dst = edge_index[1]
    batch3 = batch.reshape(NB_N, 1, BN).astype(jnp.int32)

    W1u = W1a[:D] - W1a[D:]
    W1v = W1a[D:]
    W2u = W2a[:D] - W2a[D:]
    W2v = W2a[D:]
    r = lambda b: b.reshape(1, -1)

    z128 = jnp.zeros((N, D), jnp.float32)
    ones128 = jnp.ones((CHUNK, D), jnp.float32)

    u1, v1, sc1 = _prep(x, W1u, W1v, Wsc1, r(bsc1))
    pu1, pv1 = _sc_gather(u1, v1, dst, src)
    t1 = _edge_mlp(pu1, pv1, r(b1a), W1b, r(b1b))
    sparts1 = _sc_scatter(t1, dst, z128)
    cparts = _sc_count(dst, z128, ones128)
    o1, s1, q1, c1 = _combine_stats(sparts1, cparts, sc1, batch3)

    u2, v2, sc2 = _norm_prep(o1, batch3, s1, q1, c1, W2u, W2v, Wsc2, r(bsc2))
    pu2, pv2 = _sc_gather(u2, v2, dst, src)
    t2 = _edge_mlp(pu2, pv2, r(b2a), W2b, r(b2b))
    sparts2 = _sc_scatter(t2, dst, z128)
    o2, s2, q2, c2 = _combine_stats(sparts2, cparts, sc2, batch3)

    pool = _norm_pool(o2, batch3, s2, q2, c2)
    return _final_mlp(pool, c2, Wm, r(bm), Wo, r(bo))


# speedup vs baseline: 3.9405x; 1.4037x over previous
import functools

import jax
import jax.numpy as jnp
from jax import lax
from jax.experimental import pallas as pl
from jax.experimental.pallas import tpu as pltpu
from jax.experimental.pallas import tpu_sc as plsc

N = 10000
E = 320000
D = 128
G = 64

NB_N = 10
BN = N // NB_N
BE = 2000
NB_E = E // BE

NC = 2
NS = 16
NW = NC * NS
EPW = E // NW
CHUNK = 80
CPW = EPW // CHUNK
ROWS_MAIN = 624
CW = 16
TAIL0 = NS * ROWS_MAIN
TAIL = N - TAIL0


def _prep_body(x_ref, wu_ref, wv_ref, wsc_ref, bsc_ref, u_ref, v_ref, sc_ref):
    xb = x_ref[...]
    u_ref[...] = jnp.dot(xb, wu_ref[...], preferred_element_type=jnp.float32)
    v_ref[...] = jnp.dot(xb, wv_ref[...], preferred_element_type=jnp.float32)
    sc_ref[...] = (
        jnp.dot(xb, wsc_ref[...], preferred_element_type=jnp.float32)
        + bsc_ref[...]
    )


def _edge_body(pu_ref, pv_ref, ba_ref, wb_ref, bb_ref, t_ref):
    p = jnp.maximum(pu_ref[...] + pv_ref[...] + ba_ref[...], 0.0)
    t = jnp.dot(p, wb_ref[...], preferred_element_type=jnp.float32) + bb_ref[...]
    t_ref[...] = jnp.maximum(t, 0.0)


def _combine_body(sp_ref, cp_ref, scin_ref, batch_ref,
                  o_ref, s_ref, q_ref, c_ref):
    i = pl.program_id(0)

    @pl.when(i == 0)
    def _():
        s_ref[...] = jnp.zeros_like(s_ref)
        q_ref[...] = jnp.zeros_like(q_ref)
        c_ref[...] = jnp.zeros_like(c_ref)

    sp = sp_ref[...]
    cp = cp_ref[...]
    cnt = cp[0, :, 0:1] + cp[1, :, 0:1]
    agg = (sp[0] + sp[1]) / jnp.maximum(cnt, 1.0)
    o = agg + scin_ref[...]
    o_ref[...] = o

    b = batch_ref[0, 0, :]
    gid = lax.broadcasted_iota(jnp.int32, (BN, G), 1)
    oh = (b[:, None] == gid).astype(jnp.float32)
    dn = (((0,), (0,)), ((), ()))
    s_ref[...] += lax.dot_general(oh, o, dn, preferred_element_type=jnp.float32)
    q_ref[...] += lax.dot_general(oh, o * o, dn,
                                  preferred_element_type=jnp.float32)
    colsum = jnp.sum(oh, axis=0)
    c_ref[...] += jnp.broadcast_to(colsum[None, :], (8, G))


def _norm_stats(s_ref, q_ref, c_ref):
    cg = jnp.maximum(c_ref[0, :], 1.0)[:, None]
    mean = s_ref[...] / cg
    var = jnp.maximum(q_ref[...] / cg - mean * mean, 0.0)
    return mean, var


def _norm_rows(o, batch_row, mean, var):
    gid = lax.broadcasted_iota(jnp.int32, (o.shape[0], G), 1)
    oh = (batch_row[:, None] == gid).astype(jnp.float32)
    mean_r = jnp.dot(oh, mean, preferred_element_type=jnp.float32)
    var_r = jnp.dot(oh, var, preferred_element_type=jnp.float32)
    h = (o - mean_r) * lax.rsqrt(var_r + 1e-5)
    return jnp.maximum(h, 0.0), oh


def _norm_prep_body(o_ref, batch_ref, s_ref, q_ref, c_ref,
                    wu_ref, wv_ref, wsc_ref, bsc_ref,
                    u_ref, v_ref, sc_ref):
    mean, var = _norm_stats(s_ref, q_ref, c_ref)
    h, _ = _norm_rows(o_ref[...], batch_ref[0, 0, :], mean, var)
    u_ref[...] = jnp.dot(h, wu_ref[...], preferred_element_type=jnp.float32)
    v_ref[...] = jnp.dot(h, wv_ref[...], preferred_element_type=jnp.float32)
    sc_ref[...] = (
        jnp.dot(h, wsc_ref[...], preferred_element_type=jnp.float32)
        + bsc_ref[...]
    )


def _norm_pool_body(o_ref, batch_ref, s_ref, q_ref, c_ref, pool_ref):
    i = pl.program_id(0)

    @pl.when(i == 0)
    def _():
        pool_ref[...] = jnp.zeros_like(pool_ref)

    mean, var = _norm_stats(s_ref, q_ref, c_ref)
    h, oh = _norm_rows(o_ref[...], batch_ref[0, 0, :], mean, var)
    dn = (((0,), (0,)), ((), ()))
    pool_ref[...] += lax.dot_general(oh, h, dn,
                                     preferred_element_type=jnp.float32)


def _final_body(pool_ref, c_ref, wm_ref, bm_ref, wo_ref, bo_ref, out_ref):
    cg = jnp.maximum(c_ref[0, :], 1.0)[:, None]
    pooled = pool_ref[...] / cg
    h = jnp.maximum(
        jnp.dot(pooled, wm_ref[...], preferred_element_type=jnp.float32)
        + bm_ref[...], 0.0)
    out_ref[...] = (
        jnp.dot(h, wo_ref[...], preferred_element_type=jnp.float32)
        + bo_ref[...]
    )


_full = lambda *shape: pl.BlockSpec(shape, lambda i: tuple(0 for _ in shape))


def _prep(x, wu, wv, wsc, bsc):
    return pl.pallas_call(
        _prep_body,
        grid=(NB_N,),
        in_specs=[
            pl.BlockSpec((BN, D), lambda i: (i, 0)),
            _full(D, D), _full(D, D), _full(D, D), _full(1, D),
        ],
        out_specs=[pl.BlockSpec((BN, D), lambda i: (i, 0))] * 3,
        out_shape=[jax.ShapeDtypeStruct((N, D), jnp.float32)] * 3,
    )(x, wu, wv, wsc, bsc)


def _edge_mlp(pu, pv, ba, wb, bb):
    return pl.pallas_call(
        _edge_body,
        grid=(NB_E,),
        in_specs=[
            pl.BlockSpec((BE, D), lambda i: (i, 0)),
            pl.BlockSpec((BE, D), lambda i: (i, 0)),
            _full(1, D), _full(D, D), _full(1, D),
        ],
        out_specs=pl.BlockSpec((BE, D), lambda i: (i, 0)),
        out_shape=jax.ShapeDtypeStruct((E, D), jnp.float32),
    )(pu, pv, ba, wb, bb)


def _combine_stats(sparts, cparts, scin, batch3):
    return pl.pallas_call(
        _combine_body,
        grid=(NB_N,),
        in_specs=[
            pl.BlockSpec((2, BN, D), lambda i: (0, i, 0)),
            pl.BlockSpec((2, BN, CW), lambda i: (0, i, 0)),
            pl.BlockSpec((BN, D), lambda i: (i, 0)),
            pl.BlockSpec((1, 1, BN), lambda i: (i, 0, 0)),
        ],
        out_specs=[
            pl.BlockSpec((BN, D), lambda i: (i, 0)),
            _full(G, D), _full(G, D), _full(8, G),
        ],
        out_shape=[
            jax.ShapeDtypeStruct((N, D), jnp.float32),
            jax.ShapeDtypeStruct((G, D), jnp.float32),
            jax.ShapeDtypeStruct((G, D), jnp.float32),
            jax.ShapeDtypeStruct((8, G), jnp.float32),
        ],
    )(sparts, cparts, scin, batch3)


def _norm_prep(o, batch3, s, q, c, wu, wv, wsc, bsc):
    return pl.pallas_call(
        _norm_prep_body,
        grid=(NB_N,),
        in_specs=[
            pl.BlockSpec((BN, D), lambda i: (i, 0)),
            pl.BlockSpec((1, 1, BN), lambda i: (i, 0, 0)),
            _full(G, D), _full(G, D), _full(8, G),
            _full(D, D), _full(D, D), _full(D, D), _full(1, D),
        ],
        out_specs=[pl.BlockSpec((BN, D), lambda i: (i, 0))] * 3,
        out_shape=[jax.ShapeDtypeStruct((N, D), jnp.float32)] * 3,
    )(o, batch3, s, q, c, wu, wv, wsc, bsc)


def _norm_pool(o, batch3, s, q, c):
    return pl.pallas_call(
        _norm_pool_body,
        grid=(NB_N,),
        in_specs=[
            pl.BlockSpec((BN, D), lambda i: (i, 0)),
            pl.BlockSpec((1, 1, BN), lambda i: (i, 0, 0)),
            _full(G, D), _full(G, D), _full(8, G),
        ],
        out_specs=_full(G, D),
        out_shape=jax.ShapeDtypeStruct((G, D), jnp.float32),
    )(o, batch3, s, q, c)


def _final_mlp(pool, c, wm, bm, wo, bo):
    return pl.pallas_call(
        _final_body,
        in_specs=[
            pl.BlockSpec((G, D), lambda: (0, 0)),
            pl.BlockSpec((8, G), lambda: (0, 0)),
            pl.BlockSpec((D, 256), lambda: (0, 0)),
            pl.BlockSpec((1, 256), lambda: (0, 0)),
            pl.BlockSpec((256, 2), lambda: (0, 0)),
            pl.BlockSpec((1, 2), lambda: (0, 0)),
        ],
        out_specs=pl.BlockSpec((G, 2), lambda: (0, 0)),
        out_shape=jax.ShapeDtypeStruct((G, 2), jnp.float32),
    )(pool, c, wm, bm, wo, bo)


@functools.cache
def _sc_mesh():
    return plsc.VectorSubcoreMesh(core_axis_name="c", subcore_axis_name="s")


@functools.cache
def _make_sc_gather():
    def body(u_hbm, v_hbm, dst_hbm, src_hbm, pu_hbm, pv_hbm,
             ibuf_d, ibuf_s, rows_u, rows_v,
             sem_i0, sem_i1, sem_g, sem_w0, sem_w1):
        wid = lax.axis_index("s") * NC + lax.axis_index("c")
        e0 = wid * EPW
        sem_i = (sem_i0, sem_i1)
        sem_w = (sem_w0, sem_w1)

        def idx_start(k, p):
            base = e0 + k * CHUNK
            pltpu.async_copy(dst_hbm.at[pl.ds(base, CHUNK)], ibuf_d.at[p],
                             sem_i[p])
            pltpu.async_copy(src_hbm.at[pl.ds(base, CHUNK)], ibuf_s.at[p],
                             sem_i[p])

        def idx_wait(p):
            pltpu.make_async_copy(dst_hbm.at[pl.ds(0, CHUNK)], ibuf_d.at[p],
                                  sem_i[p]).wait()
            pltpu.make_async_copy(src_hbm.at[pl.ds(0, CHUNK)], ibuf_s.at[p],
                                  sem_i[p]).wait()

        def gathers(p):
            pltpu.async_copy(u_hbm.at[ibuf_d.at[p]], rows_u.at[p],
                             sem_g).wait()
            pltpu.async_copy(v_hbm.at[ibuf_s.at[p]], rows_v.at[p],
                             sem_g).wait()

        def write_start(k, p):
            base = e0 + k * CHUNK
            pltpu.async_copy(rows_u.at[p], pu_hbm.at[pl.ds(base, CHUNK)],
                             sem_w[p])
            pltpu.async_copy(rows_v.at[p], pv_hbm.at[pl.ds(base, CHUNK)],
                             sem_w[p])

        def write_wait(p):
            pltpu.make_async_copy(rows_u.at[p], pu_hbm.at[pl.ds(0, CHUNK)],
                                  sem_w[p]).wait()
            pltpu.make_async_copy(rows_v.at[p], pv_hbm.at[pl.ds(0, CHUNK)],
                                  sem_w[p]).wait()

        idx_start(0, 0)
        idx_wait(0)
        idx_start(1, 1)
        gathers(0)
        write_start(0, 0)

        def loop(j, _):
            k = 1 + j * 2

            idx_wait(1)

            @pl.when(k + 1 < CPW)
            def _():
                idx_start(k + 1, 0)

            @pl.when(j > 0)
            def _():
                write_wait(1)

            gathers(1)
            write_start(k, 1)

            idx_wait(0)

            @pl.when(k + 2 < CPW)
            def _():
                idx_start(k + 2, 1)

            write_wait(0)
            gathers(0)
            write_start(k + 1, 0)
            return 0

        lax.fori_loop(0, (CPW - 1) // 2, loop, 0)
        write_wait(1)
        write_wait(0)

    return pl.kernel(
        body,
        out_type=[
            jax.ShapeDtypeStruct((E, D), jnp.float32),
            jax.ShapeDtypeStruct((E, D), jnp.float32),
        ],
        mesh=_sc_mesh(),
        scratch_types=[
            pltpu.VMEM((2, CHUNK), jnp.int32),
            pltpu.VMEM((2, CHUNK), jnp.int32),
            pltpu.VMEM((2, CHUNK, D), jnp.float32),
            pltpu.VMEM((2, CHUNK, D), jnp.float32),
            pltpu.SemaphoreType.DMA,
            pltpu.SemaphoreType.DMA,
            pltpu.SemaphoreType.DMA,
            pltpu.SemaphoreType.DMA,
            pltpu.SemaphoreType.DMA,
        ],
    )


def _sc_gather(u, v, dst, src):
    return _make_sc_gather()(u, v, dst, src)


@functools.cache
def _make_sc_scatter():
    def body(t_hbm, dst_hbm, z128, sparts, acc, ibuf0, ibuf1, tbuf,
             sem0, sem1):
        cid = lax.axis_index("c")
        sid = lax.axis_index("s")
        wid = sid * NC + cid
        r0 = pl.multiple_of(sid * ROWS_MAIN, 8)

        pltpu.sync_copy(z128.at[pl.ds(r0, ROWS_MAIN)],
                        acc.at[pl.ds(r0, ROWS_MAIN)])

        @pl.when(sid == NS - 1)
        def _():
            pltpu.sync_copy(z128.at[pl.ds(TAIL0, TAIL)],
                            acc.at[pl.ds(TAIL0, TAIL)])

        plsc.subcore_barrier()
        e0 = wid * EPW
        ibuf = (ibuf0, ibuf1)
        sem = (sem0, sem1)

        def load_start(k, p):
            base = e0 + k * CHUNK
            pltpu.async_copy(dst_hbm.at[pl.ds(base, CHUNK)], ibuf[p], sem[p])
            pltpu.async_copy(t_hbm.at[pl.ds(base, CHUNK)], tbuf.at[p], sem[p])

        def load_wait(p):
            pltpu.make_async_copy(dst_hbm.at[pl.ds(0, CHUNK)], ibuf[p],
                                  sem[p]).wait()
            pltpu.make_async_copy(t_hbm.at[pl.ds(0, CHUNK)], tbuf.at[p],
                                  sem[p]).wait()

        def scatter_add(p):
            pltpu.sync_copy(tbuf.at[p], acc.at[ibuf[p]], add=True)

        load_start(0, 0)
        load_wait(0)
        load_start(1, 1)
        scatter_add(0)

        def loop(j, _):
            k = 1 + j * 2

            load_wait(1)

            @pl.when(k + 1 < CPW)
            def _():
                load_start(k + 1, 0)

            scatter_add(1)
            load_wait(0)

            @pl.when(k + 2 < CPW)
            def _():
                load_start(k + 2, 1)

            scatter_add(0)
            return 0

        lax.fori_loop(0, (CPW - 1) // 2, loop, 0)
        plsc.subcore_barrier()

        o0 = pl.multiple_of(cid * N + r0, 8)
        pltpu.sync_copy(acc.at[pl.ds(r0, ROWS_MAIN)],
                        sparts.at[pl.ds(o0, ROWS_MAIN)])

        @pl.when(sid == NS - 1)
        def _():
            ot = pl.multiple_of(cid * N + TAIL0, 8)
            pltpu.sync_copy(acc.at[pl.ds(TAIL0, TAIL)],
                            sparts.at[pl.ds(ot, TAIL)])

    return pl.kernel(
        body,
        out_type=[jax.ShapeDtypeStruct((NC * N, D), jnp.float32)],
        mesh=_sc_mesh(),
        scratch_types=[
            pltpu.VMEM_SHARED((N, D), jnp.float32),
            pltpu.VMEM((CHUNK,), jnp.int32),
            pltpu.VMEM((CHUNK,), jnp.int32),
            pltpu.VMEM((2, CHUNK, D), jnp.float32),
            pltpu.SemaphoreType.DMA,
            pltpu.SemaphoreType.DMA,
        ],
    )


@functools.cache
def _make_sc_count():
    def body(dst_hbm, z128, ones_hbm, cparts, acc, ibuf0, ibuf1, ones_v,
             sem0, sem1):
        cid = lax.axis_index("c")
        sid = lax.axis_index("s")
        wid = sid * NC + cid
        r0 = pl.multiple_of(sid * ROWS_MAIN, 8)

        pltpu.sync_copy(z128.at[pl.ds(r0, ROWS_MAIN)],
                        acc.at[pl.ds(r0, ROWS_MAIN)])
        pltpu.sync_copy(ones_hbm, ones_v)

        @pl.when(sid == NS - 1)
        def _():
            pltpu.sync_copy(z128.at[pl.ds(TAIL0, TAIL)],
                            acc.at[pl.ds(TAIL0, TAIL)])

        plsc.subcore_barrier()
        e0 = wid * EPW
        ibuf = (ibuf0, ibuf1)
        sem = (sem0, sem1)

        def load_start(k, p):
            pltpu.async_copy(dst_hbm.at[pl.ds(e0 + k * CHUNK, CHUNK)],
                             ibuf[p], sem[p])

        def load_wait(p):
            pltpu.make_async_copy(dst_hbm.at[pl.ds(0, CHUNK)], ibuf[p],
                                  sem[p]).wait()

        def scatter_add(p):
            pltpu.sync_copy(ones_v, acc.at[ibuf[p]], add=True)

        load_start(0, 0)
        load_wait(0)
        load_start(1, 1)
        scatter_add(0)

        def loop(j, _):
            k = 1 + j * 2
            load_wait(1)

            @pl.when(k + 1 < CPW)
            def _():
                load_start(k + 1, 0)

            scatter_add(1)
            load_wait(0)

            @pl.when(k + 2 < CPW)
            def _():
                load_start(k + 2, 1)

            scatter_add(0)
            return 0

        lax.fori_loop(0, (CPW - 1) // 2, loop, 0)
        plsc.subcore_barrier()

        o0 = pl.multiple_of(cid * N + r0, 8)
        pltpu.sync_copy(acc.at[pl.ds(r0, ROWS_MAIN)],
                        cparts.at[pl.ds(o0, ROWS_MAIN)])

        @pl.when(sid == NS - 1)
        def _():
            ot = pl.multiple_of(cid * N + TAIL0, 8)
            pltpu.sync_copy(acc.at[pl.ds(TAIL0, TAIL)],
                            cparts.at[pl.ds(ot, TAIL)])

    return pl.kernel(
        body,
        out_type=[jax.ShapeDtypeStruct((NC * N, D), jnp.float32)],
        mesh=_sc_mesh(),
        scratch_types=[
            pltpu.VMEM_SHARED((N, D), jnp.float32),
            pltpu.VMEM((CHUNK,), jnp.int32),
            pltpu.VMEM((CHUNK,), jnp.int32),
            pltpu.VMEM((CHUNK, D), jnp.float32),
            pltpu.SemaphoreType.DMA,
            pltpu.SemaphoreType.DMA,
        ],
    )


def _sc_count(dst, z128, ones128):
    (cparts,) = _make_sc_count()(dst, z128, ones128)
    return cparts.reshape(NC, N, D)[:, :, :CW]


def _sc_scatter(t, dst, z128):
    (sparts,) = _make_sc_scatter()(t, dst, z128)
    return sparts.reshape(NC, N, D)


def kernel(x, edge_index, batch, W1a, b1a, W1b, b1b, Wsc1, bsc1,
           W2a, b2a, W2b, b2b, Wsc2, bsc2, Wm, bm, Wo, bo):
    src = edge_index[0]
    dst = edge_index[1]
    batch3 = batch.reshape(NB_N, 1, BN).astype(jnp.int32)

    W1u = W1a[:D] - W1a[D:]
    W1v = W1a[D:]
    W2u = W2a[:D] - W2a[D:]
    W2v = W2a[D:]
    r = lambda b: b.reshape(1, -1)

    z128 = jnp.zeros((N, D), jnp.float32)
    ones128 = jnp.ones((CHUNK, D), jnp.float32)

    u1, v1, sc1 = _prep(x, W1u, W1v, Wsc1, r(bsc1))
    pu1, pv1 = _sc_gather(u1, v1, dst, src)
    t1 = _edge_mlp(pu1, pv1, r(b1a), W1b, r(b1b))
    sparts1 = _sc_scatter(t1, dst, z128)
    cparts = _sc_count(dst, z128, ones128)
    o1, s1, q1, c1 = _combine_stats(sparts1, cparts, sc1, batch3)

    u2, v2, sc2 = _norm_prep(o1, batch3, s1, q1, c1, W2u, W2v, Wsc2, r(bsc2))
    pu2, pv2 = _sc_gather(u2, v2, dst, src)
    t2 = _edge_mlp(pu2, pv2, r(b2a), W2b, r(b2b))
    sparts2 = _sc_scatter(t2, dst, z128)
    o2, s2, q2, c2 = _combine_stats(sparts2, cparts, sc2, batch3)

    pool = _norm_pool(o2, batch3, s2, q2, c2)
    return _final_mlp(pool, c2, Wm, r(bm), Wo, r(bo))


# speedup vs baseline: 4.2367x; 1.0752x over previous
import functools

import jax
import jax.numpy as jnp
from jax import lax
from jax.experimental import pallas as pl
from jax.experimental.pallas import tpu as pltpu
from jax.experimental.pallas import tpu_sc as plsc

N = 10000
E = 320000
D = 128
G = 64

NB_N = 10
BN = N // NB_N
BE = 2000
NB_E = E // BE

NC = 2
NS = 16
NW = NC * NS
EPW = E // NW
CHUNK = 80
CPW = EPW // CHUNK
ROWS_MAIN = 624
CW = 16
TAIL0 = NS * ROWS_MAIN
TAIL = N - TAIL0


def _prep_body(x_ref, wu_ref, wv_ref, wsc_ref, bsc_ref, u_ref, v_ref, sc_ref):
    xb = x_ref[...]
    u_ref[...] = jnp.dot(xb, wu_ref[...], preferred_element_type=jnp.float32)
    v_ref[...] = jnp.dot(xb, wv_ref[...], preferred_element_type=jnp.float32)
    sc_ref[...] = (
        jnp.dot(xb, wsc_ref[...], preferred_element_type=jnp.float32)
        + bsc_ref[...]
    )


def _edge_body(p_ref, ba_ref, wb_ref, bb_ref, t_ref):
    p = jnp.maximum(p_ref[...] + ba_ref[...], 0.0)
    t = jnp.dot(p, wb_ref[...], preferred_element_type=jnp.float32) + bb_ref[...]
    t_ref[...] = jnp.maximum(t, 0.0)


def _combine_body(sp_ref, cp_ref, scin_ref, batch_ref,
                  o_ref, s_ref, q_ref, c_ref):
    i = pl.program_id(0)

    @pl.when(i == 0)
    def _():
        s_ref[...] = jnp.zeros_like(s_ref)
        q_ref[...] = jnp.zeros_like(q_ref)
        c_ref[...] = jnp.zeros_like(c_ref)

    sp = sp_ref[...]
    cp = cp_ref[...]
    cnt = cp[0, :, 0:1] + cp[1, :, 0:1]
    agg = (sp[0] + sp[1]) / jnp.maximum(cnt, 1.0)
    o = agg + scin_ref[...]
    o_ref[...] = o

    b = batch_ref[0, 0, :]
    gid = lax.broadcasted_iota(jnp.int32, (BN, G), 1)
    oh = (b[:, None] == gid).astype(jnp.float32)
    dn = (((0,), (0,)), ((), ()))
    s_ref[...] += lax.dot_general(oh, o, dn, preferred_element_type=jnp.float32)
    q_ref[...] += lax.dot_general(oh, o * o, dn,
                                  preferred_element_type=jnp.float32)
    colsum = jnp.sum(oh, axis=0)
    c_ref[...] += jnp.broadcast_to(colsum[None, :], (8, G))


def _norm_stats(s_ref, q_ref, c_ref):
    cg = jnp.maximum(c_ref[0, :], 1.0)[:, None]
    mean = s_ref[...] / cg
    var = jnp.maximum(q_ref[...] / cg - mean * mean, 0.0)
    return mean, var


def _norm_rows(o, batch_row, mean, var):
    gid = lax.broadcasted_iota(jnp.int32, (o.shape[0], G), 1)
    oh = (batch_row[:, None] == gid).astype(jnp.float32)
    mean_r = jnp.dot(oh, mean, preferred_element_type=jnp.float32)
    var_r = jnp.dot(oh, var, preferred_element_type=jnp.float32)
    h = (o - mean_r) * lax.rsqrt(var_r + 1e-5)
    return jnp.maximum(h, 0.0), oh


def _norm_prep_body(o_ref, batch_ref, s_ref, q_ref, c_ref,
                    wu_ref, wv_ref, wsc_ref, bsc_ref,
                    u_ref, v_ref, sc_ref):
    mean, var = _norm_stats(s_ref, q_ref, c_ref)
    h, _ = _norm_rows(o_ref[...], batch_ref[0, 0, :], mean, var)
    u_ref[...] = jnp.dot(h, wu_ref[...], preferred_element_type=jnp.float32)
    v_ref[...] = jnp.dot(h, wv_ref[...], preferred_element_type=jnp.float32)
    sc_ref[...] = (
        jnp.dot(h, wsc_ref[...], preferred_element_type=jnp.float32)
        + bsc_ref[...]
    )


def _norm_pool_body(o_ref, batch_ref, s_ref, q_ref, c_ref, pool_ref):
    i = pl.program_id(0)

    @pl.when(i == 0)
    def _():
        pool_ref[...] = jnp.zeros_like(pool_ref)

    mean, var = _norm_stats(s_ref, q_ref, c_ref)
    h, oh = _norm_rows(o_ref[...], batch_ref[0, 0, :], mean, var)
    dn = (((0,), (0,)), ((), ()))
    pool_ref[...] += lax.dot_general(oh, h, dn,
                                     preferred_element_type=jnp.float32)


def _final_body(pool_ref, c_ref, wm_ref, bm_ref, wo_ref, bo_ref, out_ref):
    cg = jnp.maximum(c_ref[0, :], 1.0)[:, None]
    pooled = pool_ref[...] / cg
    h = jnp.maximum(
        jnp.dot(pooled, wm_ref[...], preferred_element_type=jnp.float32)
        + bm_ref[...], 0.0)
    out_ref[...] = (
        jnp.dot(h, wo_ref[...], preferred_element_type=jnp.float32)
        + bo_ref[...]
    )


_full = lambda *shape: pl.BlockSpec(shape, lambda i: tuple(0 for _ in shape))


def _prep(x, wu, wv, wsc, bsc):
    return pl.pallas_call(
        _prep_body,
        grid=(NB_N,),
        in_specs=[
            pl.BlockSpec((BN, D), lambda i: (i, 0)),
            _full(D, D), _full(D, D), _full(D, D), _full(1, D),
        ],
        out_specs=[pl.BlockSpec((BN, D), lambda i: (i, 0))] * 3,
        out_shape=[jax.ShapeDtypeStruct((N, D), jnp.float32)] * 3,
    )(x, wu, wv, wsc, bsc)


def _edge_mlp(p, ba, wb, bb):
    return pl.pallas_call(
        _edge_body,
        grid=(NB_E,),
        in_specs=[
            pl.BlockSpec((BE, D), lambda i: (i, 0)),
            _full(1, D), _full(D, D), _full(1, D),
        ],
        out_specs=pl.BlockSpec((BE, D), lambda i: (i, 0)),
        out_shape=jax.ShapeDtypeStruct((E, D), jnp.float32),
    )(p, ba, wb, bb)


def _combine_stats(sparts, cparts, scin, batch3):
    return pl.pallas_call(
        _combine_body,
        grid=(NB_N,),
        in_specs=[
            pl.BlockSpec((2, BN, D), lambda i: (0, i, 0)),
            pl.BlockSpec((2, BN, CW), lambda i: (0, i, 0)),
            pl.BlockSpec((BN, D), lambda i: (i, 0)),
            pl.BlockSpec((1, 1, BN), lambda i: (i, 0, 0)),
        ],
        out_specs=[
            pl.BlockSpec((BN, D), lambda i: (i, 0)),
            _full(G, D), _full(G, D), _full(8, G),
        ],
        out_shape=[
            jax.ShapeDtypeStruct((N, D), jnp.float32),
            jax.ShapeDtypeStruct((G, D), jnp.float32),
            jax.ShapeDtypeStruct((G, D), jnp.float32),
            jax.ShapeDtypeStruct((8, G), jnp.float32),
        ],
    )(sparts, cparts, scin, batch3)


def _norm_prep(o, batch3, s, q, c, wu, wv, wsc, bsc):
    return pl.pallas_call(
        _norm_prep_body,
        grid=(NB_N,),
        in_specs=[
            pl.BlockSpec((BN, D), lambda i: (i, 0)),
            pl.BlockSpec((1, 1, BN), lambda i: (i, 0, 0)),
            _full(G, D), _full(G, D), _full(8, G),
            _full(D, D), _full(D, D), _full(D, D), _full(1, D),
        ],
        out_specs=[pl.BlockSpec((BN, D), lambda i: (i, 0))] * 3,
        out_shape=[jax.ShapeDtypeStruct((N, D), jnp.float32)] * 3,
    )(o, batch3, s, q, c, wu, wv, wsc, bsc)


def _norm_pool(o, batch3, s, q, c):
    return pl.pallas_call(
        _norm_pool_body,
        grid=(NB_N,),
        in_specs=[
            pl.BlockSpec((BN, D), lambda i: (i, 0)),
            pl.BlockSpec((1, 1, BN), lambda i: (i, 0, 0)),
            _full(G, D), _full(G, D), _full(8, G),
        ],
        out_specs=_full(G, D),
        out_shape=jax.ShapeDtypeStruct((G, D), jnp.float32),
    )(o, batch3, s, q, c)


def _final_mlp(pool, c, wm, bm, wo, bo):
    return pl.pallas_call(
        _final_body,
        in_specs=[
            pl.BlockSpec((G, D), lambda: (0, 0)),
            pl.BlockSpec((8, G), lambda: (0, 0)),
            pl.BlockSpec((D, 256), lambda: (0, 0)),
            pl.BlockSpec((1, 256), lambda: (0, 0)),
            pl.BlockSpec((256, 2), lambda: (0, 0)),
            pl.BlockSpec((1, 2), lambda: (0, 0)),
        ],
        out_specs=pl.BlockSpec((G, 2), lambda: (0, 0)),
        out_shape=jax.ShapeDtypeStruct((G, 2), jnp.float32),
    )(pool, c, wm, bm, wo, bo)


@functools.cache
def _sc_mesh():
    return plsc.VectorSubcoreMesh(core_axis_name="c", subcore_axis_name="s")


@functools.cache
def _make_sc_gather():
    def body(u_hbm, v_hbm, dst_hbm, src_hbm, p_hbm,
             ibuf_d, ibuf_s, rows_u, rows_v,
             sem_i0, sem_i1, sem_g, sem_w0, sem_w1):
        wid = lax.axis_index("s") * NC + lax.axis_index("c")
        e0 = wid * EPW
        sem_i = (sem_i0, sem_i1)
        sem_w = (sem_w0, sem_w1)

        def idx_start(k, p):
            base = e0 + k * CHUNK
            pltpu.async_copy(dst_hbm.at[pl.ds(base, CHUNK)], ibuf_d.at[p],
                             sem_i[p])
            pltpu.async_copy(src_hbm.at[pl.ds(base, CHUNK)], ibuf_s.at[p],
                             sem_i[p])

        def idx_wait(p):
            pltpu.make_async_copy(dst_hbm.at[pl.ds(0, CHUNK)], ibuf_d.at[p],
                                  sem_i[p]).wait()
            pltpu.make_async_copy(src_hbm.at[pl.ds(0, CHUNK)], ibuf_s.at[p],
                                  sem_i[p]).wait()

        def write_start(k, p):
            base = e0 + k * CHUNK
            pltpu.async_copy(rows_u.at[p], p_hbm.at[pl.ds(base, CHUNK)],
                             sem_w[p])

        def write_wait(p):
            pltpu.make_async_copy(rows_u.at[p], p_hbm.at[pl.ds(0, CHUNK)],
                                  sem_w[p]).wait()

        def add_rows(p):
            def rbody(r, _):
                for c in range(D // 16):
                    sl = pl.ds(c * 16, 16)
                    rows_u[p, r, sl] = rows_u[p, r, sl] + rows_v[p, r, sl]
                return 0

            lax.fori_loop(0, CHUNK, rbody, 0)

        def gather_u(k, p):
            return pltpu.async_copy(u_hbm.at[ibuf_d.at[p]], rows_u.at[p],
                                    sem_g)

        def gather_v(k, p):
            return pltpu.async_copy(v_hbm.at[ibuf_s.at[p]], rows_v.at[p],
                                    sem_g)

        idx_start(0, 0)
        idx_wait(0)
        idx_start(1, 1)
        gather_u(0, 0).wait()
        gather_v(0, 0).wait()

        def step(j, k, p):
            q = 1 - p
            idx_wait(q)

            @pl.when(k + 2 < CPW)
            def _():
                idx_start(k + 2, p)

            if p == 0:
                @pl.when(j > 0)
                def _():
                    write_wait(q)
            else:
                write_wait(q)

            cu = gather_u(k + 1, q)
            add_rows(p)
            cu.wait()
            cv = gather_v(k + 1, q)
            write_start(k, p)
            cv.wait()

        def loop(j, _):
            step(j, 2 * j, 0)
            step(j, 2 * j + 1, 1)
            return 0

        lax.fori_loop(0, (CPW - 1) // 2, loop, 0)
        add_rows(0)
        write_start(CPW - 1, 0)
        write_wait(1)
        write_wait(0)

    return pl.kernel(
        body,
        out_type=[jax.ShapeDtypeStruct((E, D), jnp.float32)],
        mesh=_sc_mesh(),
        scratch_types=[
            pltpu.VMEM((2, CHUNK), jnp.int32),
            pltpu.VMEM((2, CHUNK), jnp.int32),
            pltpu.VMEM((2, CHUNK, D), jnp.float32),
            pltpu.VMEM((2, CHUNK, D), jnp.float32),
            pltpu.SemaphoreType.DMA,
            pltpu.SemaphoreType.DMA,
            pltpu.SemaphoreType.DMA,
            pltpu.SemaphoreType.DMA,
            pltpu.SemaphoreType.DMA,
        ],
    )


def _sc_gather(u, v, dst, src):
    (p,) = _make_sc_gather()(u, v, dst, src)
    return p


@functools.cache
def _make_sc_scatter():
    def body(t_hbm, dst_hbm, z128, sparts, acc, ibuf0, ibuf1, tbuf,
             sem0, sem1):
        cid = lax.axis_index("c")
        sid = lax.axis_index("s")
        wid = sid * NC + cid
        r0 = pl.multiple_of(sid * ROWS_MAIN, 8)

        pltpu.sync_copy(z128.at[pl.ds(r0, ROWS_MAIN)],
                        acc.at[pl.ds(r0, ROWS_MAIN)])

        @pl.when(sid == NS - 1)
        def _():
            pltpu.sync_copy(z128.at[pl.ds(TAIL0, TAIL)],
                            acc.at[pl.ds(TAIL0, TAIL)])

        plsc.subcore_barrier()
        e0 = wid * EPW
        ibuf = (ibuf0, ibuf1)
        sem = (sem0, sem1)

        def load_start(k, p):
            base = e0 + k * CHUNK
            pltpu.async_copy(dst_hbm.at[pl.ds(base, CHUNK)], ibuf[p], sem[p])
            pltpu.async_copy(t_hbm.at[pl.ds(base, CHUNK)], tbuf.at[p], sem[p])

        def load_wait(p):
            pltpu.make_async_copy(dst_hbm.at[pl.ds(0, CHUNK)], ibuf[p],
                                  sem[p]).wait()
            pltpu.make_async_copy(t_hbm.at[pl.ds(0, CHUNK)], tbuf.at[p],
                                  sem[p]).wait()

        def scatter_add(p):
            pltpu.sync_copy(tbuf.at[p], acc.at[ibuf[p]], add=True)

        load_start(0, 0)
        load_wait(0)
        load_start(1, 1)
        scatter_add(0)

        def loop(j, _):
            k = 1 + j * 2

            load_wait(1)

            @pl.when(k + 1 < CPW)
            def _():
                load_start(k + 1, 0)

            scatter_add(1)
            load_wait(0)

            @pl.when(k + 2 < CPW)
            def _():
                load_start(k + 2, 1)

            scatter_add(0)
            return 0

        lax.fori_loop(0, (CPW - 1) // 2, loop, 0)
        plsc.subcore_barrier()

        o0 = pl.multiple_of(cid * N + r0, 8)
        pltpu.sync_copy(acc.at[pl.ds(r0, ROWS_MAIN)],
                        sparts.at[pl.ds(o0, ROWS_MAIN)])

        @pl.when(sid == NS - 1)
        def _():
            ot = pl.multiple_of(cid * N + TAIL0, 8)
            pltpu.sync_copy(acc.at[pl.ds(TAIL0, TAIL)],
                            sparts.at[pl.ds(ot, TAIL)])

    return pl.kernel(
        body,
        out_type=[jax.ShapeDtypeStruct((NC * N, D), jnp.float32)],
        mesh=_sc_mesh(),
        scratch_types=[
            pltpu.VMEM_SHARED((N, D), jnp.float32),
            pltpu.VMEM((CHUNK,), jnp.int32),
            pltpu.VMEM((CHUNK,), jnp.int32),
            pltpu.VMEM((2, CHUNK, D), jnp.float32),
            pltpu.SemaphoreType.DMA,
            pltpu.SemaphoreType.DMA,
        ],
    )


@functools.cache
def _make_sc_count():
    def body(dst_hbm, z128, ones_hbm, cparts, acc, ibuf0, ibuf1, ones_v,
             sem0, sem1):
        cid = lax.axis_index("c")
        sid = lax.axis_index("s")
        wid = sid * NC + cid
        r0 = pl.multiple_of(sid * ROWS_MAIN, 8)

        pltpu.sync_copy(z128.at[pl.ds(r0, ROWS_MAIN)],
                        acc.at[pl.ds(r0, ROWS_MAIN)])
        pltpu.sync_copy(ones_hbm, ones_v)

        @pl.when(sid == NS - 1)
        def _():
            pltpu.sync_copy(z128.at[pl.ds(TAIL0, TAIL)],
                            acc.at[pl.ds(TAIL0, TAIL)])

        plsc.subcore_barrier()
        e0 = wid * EPW
        ibuf = (ibuf0, ibuf1)
        sem = (sem0, sem1)

        def load_start(k, p):
            pltpu.async_copy(dst_hbm.at[pl.ds(e0 + k * CHUNK, CHUNK)],
                             ibuf[p], sem[p])

        def load_wait(p):
            pltpu.make_async_copy(dst_hbm.at[pl.ds(0, CHUNK)], ibuf[p],
                                  sem[p]).wait()

        def scatter_add(p):
            pltpu.sync_copy(ones_v, acc.at[ibuf[p]], add=True)

        load_start(0, 0)
        load_wait(0)
        load_start(1, 1)
        scatter_add(0)

        def loop(j, _):
            k = 1 + j * 2
            load_wait(1)

            @pl.when(k + 1 < CPW)
            def _():
                load_start(k + 1, 0)

            scatter_add(1)
            load_wait(0)

            @pl.when(k + 2 < CPW)
            def _():
                load_start(k + 2, 1)

            scatter_add(0)
            return 0

        lax.fori_loop(0, (CPW - 1) // 2, loop, 0)
        plsc.subcore_barrier()

        o0 = pl.multiple_of(cid * N + r0, 8)
        pltpu.sync_copy(acc.at[pl.ds(r0, ROWS_MAIN)],
                        cparts.at[pl.ds(o0, ROWS_MAIN)])

        @pl.when(sid == NS - 1)
        def _():
            ot = pl.multiple_of(cid * N + TAIL0, 8)
            pltpu.sync_copy(acc.at[pl.ds(TAIL0, TAIL)],
                            cparts.at[pl.ds(ot, TAIL)])

    return pl.kernel(
        body,
        out_type=[jax.ShapeDtypeStruct((NC * N, D), jnp.float32)],
        mesh=_sc_mesh(),
        scratch_types=[
            pltpu.VMEM_SHARED((N, D), jnp.float32),
            pltpu.VMEM((CHUNK,), jnp.int32),
            pltpu.VMEM((CHUNK,), jnp.int32),
            pltpu.VMEM((CHUNK, D), jnp.float32),
            pltpu.SemaphoreType.DMA,
            pltpu.SemaphoreType.DMA,
        ],
    )


def _sc_count(dst, z128, ones128):
    (cparts,) = _make_sc_count()(dst, z128, ones128)
    return cparts.reshape(NC, N, D)[:, :, :CW]


def _sc_scatter(t, dst, z128):
    (sparts,) = _make_sc_scatter()(t, dst, z128)
    return sparts.reshape(NC, N, D)


def kernel(x, edge_index, batch, W1a, b1a, W1b, b1b, Wsc1, bsc1,
           W2a, b2a, W2b, b2b, Wsc2, bsc2, Wm, bm, Wo, bo):
    src = edge_index[0]
    dst = edge_index[1]
    batch3 = batch.reshape(NB_N, 1, BN).astype(jnp.int32)

    W1u = W1a[:D] - W1a[D:]
    W1v = W1a[D:]
    W2u = W2a[:D] - W2a[D:]
    W2v = W2a[D:]
    r = lambda b: b.reshape(1, -1)

    z128 = jnp.zeros((N, D), jnp.float32)
    ones128 = jnp.ones((CHUNK, D), jnp.float32)

    u1, v1, sc1 = _prep(x, W1u, W1v, Wsc1, r(bsc1))
    p1 = _sc_gather(u1, v1, dst, src)
    t1 = _edge_mlp(p1, r(b1a), W1b, r(b1b))
    sparts1 = _sc_scatter(t1, dst, z128)
    cparts = _sc_count(dst, z128, ones128)
    o1, s1, q1, c1 = _combine_stats(sparts1, cparts, sc1, batch3)

    u2, v2, sc2 = _norm_prep(o1, batch3, s1, q1, c1, W2u, W2v, Wsc2, r(bsc2))
    p2 = _sc_gather(u2, v2, dst, src)
    t2 = _edge_mlp(p2, r(b2a), W2b, r(b2b))
    sparts2 = _sc_scatter(t2, dst, z128)
    o2, s2, q2, c2 = _combine_stats(sparts2, cparts, sc2, batch3)

    pool = _norm_pool(o2, batch3, s2, q2, c2)
    return _final_mlp(pool, c2, Wm, r(bm), Wo, r(bo))
